# baseline (speedup 1.0000x reference)
import functools

import jax
import jax.numpy as jnp
import numpy as np
from jax import lax
from jax.experimental import pallas as pl
from jax.experimental.pallas import tpu as pltpu
from jax.experimental.pallas import tpu_sc as plsc

ROWS = 128
N = 100000
POWER = np.float32(0.1)

NC, NS = 2, 16
NW = NC * NS
RPW = ROWS // NW
CHUNK = 10000
NCHUNK = N // CHUNK
RED_UNROLL = 10
CMB_UNROLL = 5

_PERM = np.array([
    121, 35, 45, 99, 31, 112, 85, 63, 117, 114, 82, 65, 7, 4, 101, 102,
    78, 29, 108, 83, 44, 16, 58, 123, 37, 111, 19, 61, 2, 34, 5, 90,
    110, 72, 30, 42, 3, 70, 67, 39, 56, 69, 80, 22, 6, 118, 54, 77,
    18, 10, 11, 53, 94, 32, 15, 49, 50, 20, 43, 92, 8, 24, 81, 96,
    106, 9, 40, 71, 93, 59, 75, 97, 66, 25, 73, 13, 52, 88, 62, 87,
    76, 60, 47, 33, 79, 14, 17, 38, 86, 23, 105, 0, 41, 64, 21, 124,
    116, 26, 57, 89, 126, 125, 1, 115, 28, 113, 48, 36, 119, 120, 122, 100,
    91, 55, 103, 51, 127, 98, 107, 27, 74, 12, 109, 84, 68, 104, 95, 46,
], dtype=np.int32)
_PERMW = _PERM.reshape(NW, RPW, 1).astype(np.int32)


def _body(in_hbm, permw_hbm, out_hbm, idx_v, noise_v, chunk_v, red_v, sem):
    wid = lax.axis_index("s") * NC + lax.axis_index("c")
    pltpu.sync_copy(permw_hbm.at[wid], idx_v)
    lanes = lax.iota(jnp.int32, 16)
    for r in range(RPW):
        i = wid * RPW + r
        pltpu.async_copy(in_hbm.at[idx_v.at[r]], noise_v, sem).wait()

        def _red(k, acc, _r=r):
            base = k * (16 * RED_UNROLL)
            for u in range(RED_UNROLL):
                acc = acc + noise_v[0, pl.ds(base + u * 16, 16)]
            return acc

        acc = lax.fori_loop(0, N // (16 * RED_UNROLL), _red,
                            jnp.zeros((16,), jnp.float32))
        for sh in (8, 4, 2, 1):
            red_v[...] = acc
            acc = acc + plsc.load_gather(red_v, [lanes ^ sh])
        corr = acc * np.float32(POWER / N)

        for cb in range(NCHUNK):
            pltpu.sync_copy(in_hbm.at[i, pl.ds(cb * CHUNK, CHUNK)], chunk_v)

            def _cmb(j, _, _cb=cb, _corr=corr):
                base = j * (16 * CMB_UNROLL)
                for u in range(CMB_UNROLL):
                    off = base + u * 16
                    x = chunk_v[pl.ds(off, 16)]
                    nz = noise_v[0, pl.ds(_cb * CHUNK + off, 16)]
                    chunk_v[pl.ds(off, 16)] = x + POWER * nz - _corr
                return 0

            lax.fori_loop(0, CHUNK // (16 * CMB_UNROLL), _cmb, 0)
            pltpu.sync_copy(chunk_v, out_hbm.at[i, pl.ds(cb * CHUNK, CHUNK)])


@jax.jit
def _emix_noise_sc(inpute, permw):
    fn = pl.kernel(
        _body,
        out_type=jax.ShapeDtypeStruct((ROWS, N), jnp.float32),
        mesh=plsc.VectorSubcoreMesh(core_axis_name="c", subcore_axis_name="s"),
        scratch_types=[
            pltpu.VMEM((RPW, 1), jnp.int32),
            pltpu.VMEM((1, N), jnp.float32),
            pltpu.VMEM((CHUNK,), jnp.float32),
            pltpu.VMEM((16,), jnp.float32),
            pltpu.SemaphoreType.DMA,
        ],
        compiler_params=pltpu.CompilerParams(
            use_tc_tiling_on_sc=False, needs_layout_passes=False),
    )
    return fn(inpute, permw)


def kernel(inpute):
    return _emix_noise_sc(inpute, jnp.asarray(_PERMW))


# speedup vs baseline: 1.2089x; 1.2089x over previous
import functools

import jax
import jax.numpy as jnp
import numpy as np
from jax import lax
from jax.experimental import pallas as pl
from jax.experimental.pallas import tpu as pltpu
from jax.experimental.pallas import tpu_sc as plsc

ROWS = 128
N = 100000
POWER = np.float32(0.1)

NC, NS = 2, 16
GROUP = ROWS // NS
CW = 4608
NCHUNK = 11
HALF0 = NCHUNK * CW
NMAIN = (N // 128) * 128
TAIL = N - NMAIN
_SC1_LAST = NMAIN - CW
CG = CW // 16

_PERM = np.array([
    121, 35, 45, 99, 31, 112, 85, 63, 117, 114, 82, 65, 7, 4, 101, 102,
    78, 29, 108, 83, 44, 16, 58, 123, 37, 111, 19, 61, 2, 34, 5, 90,
    110, 72, 30, 42, 3, 70, 67, 39, 56, 69, 80, 22, 6, 118, 54, 77,
    18, 10, 11, 53, 94, 32, 15, 49, 50, 20, 43, 92, 8, 24, 81, 96,
    106, 9, 40, 71, 93, 59, 75, 97, 66, 25, 73, 13, 52, 88, 62, 87,
    76, 60, 47, 33, 79, 14, 17, 38, 86, 23, 105, 0, 41, 64, 21, 124,
    116, 26, 57, 89, 126, 125, 1, 115, 28, 113, 48, 36, 119, 120, 122, 100,
    91, 55, 103, 51, 127, 98, 107, 27, 74, 12, 109, 84, 68, 104, 95, 46,
], dtype=np.int32)


_RB = 6400
_RG = -(-N // _RB)


def _sums_body(in_ref, out_ref):
    j = pl.program_id(0)

    @pl.when(j == 0)
    def _():
        out_ref[...] = jnp.zeros_like(out_ref)

    col = lax.broadcasted_iota(jnp.int32, (ROWS, _RB), 1) + j * _RB
    x = jnp.where(col < N, in_ref[...], 0.0)
    out_ref[...] += jnp.sum(x, axis=1, keepdims=True)


def _row_sums(inpute):
    return pl.pallas_call(
        _sums_body,
        grid=(_RG,),
        in_specs=[pl.BlockSpec((ROWS, _RB), lambda j: (0, j))],
        out_specs=pl.BlockSpec((ROWS, 1), lambda j: (0, 0)),
        out_shape=jax.ShapeDtypeStruct((ROWS, 1), jnp.float32),
    )(inpute)


def _sc_body(in_hbm, corr_hbm, perm_hbm, out_hbm,
             idx_v, permall_v, blk_v, noise_v, corr_v, tail_v, tout_v, sem):
    c = lax.axis_index("c")
    s = lax.axis_index("s")
    pltpu.sync_copy(perm_hbm, permall_v)
    pltpu.sync_copy(corr_hbm.at[pl.ds(s * 128, 128)], corr_v)
    lanes = lax.iota(jnp.int32, 16)
    zeros = lanes * 0
    lo8 = lanes < 8
    mine = plsc.load_gather(permall_v, [zeros + s * GROUP + (lanes & 7)])
    plsc.store_scatter(idx_v, [lanes], mine, mask=lo8)

    for j in range(NCHUNK):
        delta = HALF0 if j < NCHUNK - 1 else _SC1_LAST - (NCHUNK - 1) * CW
        off = j * CW + c * delta

        pltpu.sync_copy(in_hbm.at[pl.ds(s * GROUP, GROUP), pl.ds(off, CW)],
                        blk_v)
        pltpu.async_copy(in_hbm.at[:, pl.ds(off, CW)].at[idx_v],
                         noise_v, sem).wait()

        for k in range(GROUP):
            ck = corr_v[pl.ds(16 * k, 16)]

            def _cmb(g, _, _k=k, _ck=ck):
                base = g * 64
                for u in range(4):
                    o = base + u * 16
                    x = blk_v[_k, pl.ds(o, 16)]
                    nz = noise_v[_k, pl.ds(o, 16)]
                    blk_v[_k, pl.ds(o, 16)] = x + POWER * nz - _ck
                return 0

            lax.fori_loop(0, CG // 4, _cmb, 0)

        pltpu.sync_copy(blk_v,
                        out_hbm.at[pl.ds(s * GROUP, GROUP), pl.ds(off, CW)])

    @pl.when(c == 0)
    def _tail():
        pltpu.sync_copy(in_hbm.at[:, pl.ds(NMAIN, TAIL)], tail_v)
        for k in range(GROUP):
            jv = plsc.load_gather(permall_v, [zeros + s * GROUP + k])
            ck = corr_v[pl.ds(16 * k, 16)]
            for h in range(2):
                nz = plsc.load_gather(tail_v, [jv, lanes + 16 * h])
                x = plsc.load_gather(tail_v,
                                     [zeros + (s * GROUP + k), lanes + 16 * h])
                tout_v[k, pl.ds(16 * h, 16)] = x + POWER * nz - ck

        pltpu.sync_copy(tout_v,
                        out_hbm.at[pl.ds(s * GROUP, GROUP), pl.ds(NMAIN, TAIL)])


def _sc_combine(inpute, corrw, perm):
    fn = pl.kernel(
        _sc_body,
        out_type=jax.ShapeDtypeStruct((ROWS, N), jnp.float32),
        mesh=plsc.VectorSubcoreMesh(core_axis_name="c", subcore_axis_name="s"),
        scratch_types=[
            pltpu.VMEM((GROUP,), jnp.int32),
            pltpu.VMEM((ROWS,), jnp.int32),
            pltpu.VMEM((GROUP, CW), jnp.float32),
            pltpu.VMEM((GROUP, CW), jnp.float32),
            pltpu.VMEM((GROUP * 16,), jnp.float32),
            pltpu.VMEM((ROWS, TAIL), jnp.float32),
            pltpu.VMEM((GROUP, TAIL), jnp.float32),
            pltpu.SemaphoreType.DMA,
        ],
        compiler_params=pltpu.CompilerParams(
            use_tc_tiling_on_sc=True, needs_layout_passes=False),
    )
    return fn(inpute, corrw, perm)


@jax.jit
def _emix_noise(inpute, perm):
    sums = _row_sums(inpute)
    corr = sums[:, 0] * np.float32(POWER / N)
    corrw = jnp.broadcast_to(corr[_PERM].reshape(NS, GROUP, 1),
                             (NS, GROUP, 16)).reshape(-1)
    return _sc_combine(inpute, corrw, perm)


def kernel(inpute):
    return _emix_noise(inpute, jnp.asarray(_PERM))


# speedup vs baseline: 1.5110x; 1.2499x over previous
import functools

import jax
import jax.numpy as jnp
import numpy as np
from jax import lax
from jax.experimental import pallas as pl
from jax.experimental.pallas import tpu as pltpu
from jax.experimental.pallas import tpu_sc as plsc

ROWS = 128
N = 100000
POWER = np.float32(0.1)

NC, NS = 2, 16
GROUP = ROWS // NS
CW = 3456
NCHUNK = 15
NMAIN = (N // 128) * 128
TAIL = N - NMAIN
SC1_BASE = NMAIN - NCHUNK * CW
CG = CW // 16

_PERM = np.array([
    121, 35, 45, 99, 31, 112, 85, 63, 117, 114, 82, 65, 7, 4, 101, 102,
    78, 29, 108, 83, 44, 16, 58, 123, 37, 111, 19, 61, 2, 34, 5, 90,
    110, 72, 30, 42, 3, 70, 67, 39, 56, 69, 80, 22, 6, 118, 54, 77,
    18, 10, 11, 53, 94, 32, 15, 49, 50, 20, 43, 92, 8, 24, 81, 96,
    106, 9, 40, 71, 93, 59, 75, 97, 66, 25, 73, 13, 52, 88, 62, 87,
    76, 60, 47, 33, 79, 14, 17, 38, 86, 23, 105, 0, 41, 64, 21, 124,
    116, 26, 57, 89, 126, 125, 1, 115, 28, 113, 48, 36, 119, 120, 122, 100,
    91, 55, 103, 51, 127, 98, 107, 27, 74, 12, 109, 84, 68, 104, 95, 46,
], dtype=np.int32)
_PMAT = np.zeros((ROWS, ROWS), dtype=np.float32)
_PMAT[np.arange(ROWS), _PERM] = 1.0


_RB = 6400
_RG = -(-N // _RB)


def _corr_body(in_ref, pmat_ref, out_ref, acc_ref):
    j = pl.program_id(0)

    @pl.when(j == 0)
    def _():
        acc_ref[...] = jnp.zeros_like(acc_ref)

    col = lax.broadcasted_iota(jnp.int32, (ROWS, _RB), 1) + j * _RB
    x = jnp.where(col < N, in_ref[...], 0.0)
    acc_ref[...] += jnp.sum(x, axis=1, keepdims=True)

    @pl.when(j == _RG - 1)
    def _():
        permuted = jnp.dot(pmat_ref[...], acc_ref[...],
                           preferred_element_type=jnp.float32)
        out_ref[...] = jnp.broadcast_to(permuted * np.float32(POWER / N),
                                        (ROWS, 16))


def _corrections(inpute):
    return pl.pallas_call(
        _corr_body,
        grid=(_RG,),
        in_specs=[pl.BlockSpec((ROWS, _RB), lambda j: (0, j)),
                  pl.BlockSpec((ROWS, ROWS), lambda j: (0, 0))],
        out_specs=pl.BlockSpec((ROWS, 16), lambda j: (0, 0)),
        out_shape=jax.ShapeDtypeStruct((ROWS, 16), jnp.float32),
        scratch_shapes=[pltpu.VMEM((ROWS, 1), jnp.float32)],
    )(inpute, jnp.asarray(_PMAT))


def _sc_body(in_hbm, corr_hbm, perm_hbm, out_hbm,
             idx_v, permall_v, blk0, blk1, nz0, nz1, corr_v,
             tail_v, tout_v, si0, si1, sg0, sg1, so0, so1):
    c = lax.axis_index("c")
    s = lax.axis_index("s")
    pltpu.sync_copy(perm_hbm, permall_v)
    pltpu.sync_copy(corr_hbm.at[pl.ds(s * GROUP, GROUP), :], corr_v)
    lanes = lax.iota(jnp.int32, 16)
    zeros = lanes * 0
    lo8 = lanes < 8
    mine = plsc.load_gather(permall_v, [zeros + s * GROUP + (lanes & 7)])
    plsc.store_scatter(idx_v, [lanes], mine, mask=lo8)

    base = c * SC1_BASE
    blk = (blk0, blk1)
    nzb = (nz0, nz1)
    sis = (si0, si1)
    sgs = (sg0, sg1)
    sos = (so0, so1)

    def _in_src(j):
        return in_hbm.at[pl.ds(s * GROUP, GROUP), pl.ds(base + j * CW, CW)]

    def _nz_src(j):
        return in_hbm.at[:, pl.ds(base + j * CW, CW)].at[idx_v]

    def _out_dst(j):
        return out_hbm.at[pl.ds(s * GROUP, GROUP), pl.ds(base + j * CW, CW)]

    def _start(j, b):
        pltpu.async_copy(_in_src(j), blk[b], sis[b])
        pltpu.async_copy(_nz_src(j), nzb[b], sgs[b])

    def _chunk(j, b):
        pltpu.make_async_copy(_in_src(j), blk[b], sis[b]).wait()
        pltpu.make_async_copy(_nz_src(j), nzb[b], sgs[b]).wait()

        @pl.when(jnp.logical_and(j >= 1, j <= NCHUNK - 2))
        def _():
            pltpu.make_async_copy(blk[1 - b], _out_dst(j - 1), sos[1 - b]).wait()

        @pl.when(j <= NCHUNK - 2)
        def _():
            _start(j + 1, 1 - b)

        for k in range(GROUP):
            ck = corr_v[k, pl.ds(0, 16)]

            def _cmb(g, _, _k=k, _ck=ck, _b=b):
                o = g * 32
                for u in range(2):
                    x = blk[_b][_k, pl.ds(o + u * 16, 16)]
                    nz = nzb[_b][_k, pl.ds(o + u * 16, 16)]
                    blk[_b][_k, pl.ds(o + u * 16, 16)] = x + POWER * nz - _ck
                return 0

            lax.fori_loop(0, CG // 2, _cmb, 0)

        pltpu.async_copy(blk[b], _out_dst(j), sos[b])

    def _body(j, _):
        pl.when(j % 2 == 0)(lambda: _chunk(j, 0))
        pl.when(j % 2 == 1)(lambda: _chunk(j, 1))
        return 0

    _start(0, 0)
    lax.fori_loop(0, NCHUNK, _body, 0)
    pltpu.make_async_copy(blk[1], _out_dst(NCHUNK - 2), sos[1]).wait()
    pltpu.make_async_copy(blk[0], _out_dst(NCHUNK - 1), sos[0]).wait()

    @pl.when(c == 0)
    def _tail():
        pltpu.sync_copy(in_hbm.at[:, pl.ds(NMAIN, TAIL)], tail_v)
        for k in range(GROUP):
            jv = plsc.load_gather(permall_v, [zeros + s * GROUP + k])
            ck = corr_v[k, pl.ds(0, 16)]
            for h in range(2):
                nz = plsc.load_gather(tail_v, [jv, lanes + 16 * h])
                x = plsc.load_gather(tail_v,
                                     [zeros + (s * GROUP + k), lanes + 16 * h])
                tout_v[k, pl.ds(16 * h, 16)] = x + POWER * nz - ck

        pltpu.sync_copy(tout_v,
                        out_hbm.at[pl.ds(s * GROUP, GROUP), pl.ds(NMAIN, TAIL)])


def _sc_combine(inpute, corrw, perm):
    fn = pl.kernel(
        _sc_body,
        out_type=jax.ShapeDtypeStruct((ROWS, N), jnp.float32),
        mesh=plsc.VectorSubcoreMesh(core_axis_name="c", subcore_axis_name="s"),
        scratch_types=[
            pltpu.VMEM((GROUP,), jnp.int32),
            pltpu.VMEM((ROWS,), jnp.int32),
            pltpu.VMEM((GROUP, CW), jnp.float32),
            pltpu.VMEM((GROUP, CW), jnp.float32),
            pltpu.VMEM((GROUP, CW), jnp.float32),
            pltpu.VMEM((GROUP, CW), jnp.float32),
            pltpu.VMEM((GROUP, 16), jnp.float32),
            pltpu.VMEM((ROWS, TAIL), jnp.float32),
            pltpu.VMEM((GROUP, TAIL), jnp.float32),
            pltpu.SemaphoreType.DMA,
            pltpu.SemaphoreType.DMA,
            pltpu.SemaphoreType.DMA,
            pltpu.SemaphoreType.DMA,
            pltpu.SemaphoreType.DMA,
            pltpu.SemaphoreType.DMA,
        ],
        compiler_params=pltpu.CompilerParams(
            use_tc_tiling_on_sc=True, needs_layout_passes=False),
    )
    return fn(inpute, corrw, perm)


@jax.jit
def _emix_noise(inpute, perm):
    corrw = _corrections(inpute)
    return _sc_combine(inpute, corrw, perm)


def kernel(inpute):
    return _emix_noise(inpute, jnp.asarray(_PERM))
